# baseline (speedup 1.0000x reference)
import functools

import jax
import jax.numpy as jnp
from jax import lax
from jax.experimental import pallas as pl
from jax.experimental.pallas import tpu as pltpu
from jax.experimental.pallas import tpu_sc as plsc

B = 16384
D = 128
V = 1000
NC = 2
NS = 16
NW = NC * NS
BPW = B // NW
CHUNK = 512
NCHUNK = BPW // CHUNK

_mesh = plsc.VectorSubcoreMesh(core_axis_name="c", subcore_axis_name="s")


@functools.partial(
    pl.kernel,
    mesh=_mesh,
    out_type=jax.ShapeDtypeStruct((B, D), jnp.float32),
    scratch_types=[
        pltpu.VMEM((BPW,), jnp.int32),
        pltpu.VMEM((NCHUNK, CHUNK, D), jnp.float32),
        pltpu.VMEM_SHARED((V, D), jnp.float32),
    ]
    + [pltpu.SemaphoreType.DMA] * (NCHUNK + 1),
)
def _sc_gather2(t_hbm, table_hbm, out_hbm, idx_v, rows_v, tshared, *sems):
    gsems, ssem = sems[:NCHUNK], sems[NCHUNK]
    sid = lax.axis_index("s")
    wid = sid * NC + lax.axis_index("c")
    base = wid * BPW

    @pl.when(sid == 0)
    def _load_table():
        pltpu.sync_copy(table_hbm, tshared)

    pltpu.sync_copy(t_hbm.at[pl.ds(base, BPW)], idx_v)
    plsc.subcore_barrier()
    gcps = [
        pltpu.async_copy(
            tshared.at[idx_v.at[pl.ds(j * CHUNK, CHUNK)]], rows_v.at[j], gsems[j]
        )
        for j in range(NCHUNK)
    ]
    scps = []
    for j in range(NCHUNK):
        gcps[j].wait()
        scps.append(
            pltpu.async_copy(
                rows_v.at[j], out_hbm.at[pl.ds(base + j * CHUNK, CHUNK)], ssem
            )
        )
    for cp in scps:
        cp.wait()


def kernel(t, table):
    return _sc_gather2(t, table)


# speedup vs baseline: 1.0011x; 1.0011x over previous
import functools

import jax
import jax.numpy as jnp
from jax import lax
from jax.experimental import pallas as pl
from jax.experimental.pallas import tpu as pltpu
from jax.experimental.pallas import tpu_sc as plsc

B = 16384
D = 128
V = 1000
NC = 2
NS = 16
NW = NC * NS
BPW = B // NW
CHUNK = 256
NCHUNK = BPW // CHUNK

_mesh = plsc.VectorSubcoreMesh(core_axis_name="c", subcore_axis_name="s")


@functools.partial(
    pl.kernel,
    mesh=_mesh,
    out_type=jax.ShapeDtypeStruct((B, D), jnp.float32),
    scratch_types=[
        pltpu.VMEM((BPW,), jnp.int32),
        pltpu.VMEM((NCHUNK, CHUNK, D), jnp.float32),
        pltpu.VMEM_SHARED((V, D), jnp.float32),
    ]
    + [pltpu.SemaphoreType.DMA] * (NCHUNK + 1),
)
def _sc_gather2(t_hbm, table_hbm, out_hbm, idx_v, rows_v, tshared, *sems):
    gsems, ssem = sems[:NCHUNK], sems[NCHUNK]
    sid = lax.axis_index("s")
    wid = sid * NC + lax.axis_index("c")
    base = wid * BPW

    @pl.when(sid == 0)
    def _load_table():
        pltpu.sync_copy(table_hbm, tshared)

    pltpu.sync_copy(t_hbm.at[pl.ds(base, BPW)], idx_v)
    plsc.subcore_barrier()
    gcps = [
        pltpu.async_copy(
            tshared.at[idx_v.at[pl.ds(j * CHUNK, CHUNK)]], rows_v.at[j], gsems[j]
        )
        for j in range(NCHUNK)
    ]
    scps = []
    for j in range(NCHUNK):
        gcps[j].wait()
        scps.append(
            pltpu.async_copy(
                rows_v.at[j], out_hbm.at[pl.ds(base + j * CHUNK, CHUNK)], ssem
            )
        )
    for cp in scps:
        cp.wait()


def kernel(t, table):
    return _sc_gather2(t, table)


# speedup vs baseline: 1.0232x; 1.0221x over previous
import functools

import jax
import jax.numpy as jnp
from jax import lax
from jax.experimental import pallas as pl
from jax.experimental.pallas import tpu as pltpu
from jax.experimental.pallas import tpu_sc as plsc

B = 16384
D = 128
V = 1000
NC = 2
NS = 16
NW = NC * NS
BPW = B // NW
CHUNK = 256
NCHUNK = BPW // CHUNK

_mesh = plsc.VectorSubcoreMesh(core_axis_name="c", subcore_axis_name="s")


@functools.partial(
    pl.kernel,
    mesh=_mesh,
    out_type=jax.ShapeDtypeStruct((B, D), jnp.float32),
    scratch_types=[
        pltpu.VMEM((BPW,), jnp.int32),
        pltpu.VMEM((NCHUNK, CHUNK, D), jnp.float32),
        pltpu.VMEM_SHARED((V, D), jnp.float32),
    ]
    + [pltpu.SemaphoreType.DMA] * (NCHUNK + 2),
)
def _sc_gather2(t_hbm, table_hbm, out_hbm, idx_v, rows_v, tshared, *sems):
    gsems, ssem, isem = sems[:NCHUNK], sems[NCHUNK], sems[NCHUNK + 1]
    sid = lax.axis_index("s")
    wid = sid * NC + lax.axis_index("c")
    base = wid * BPW

    icp = pltpu.async_copy(t_hbm.at[pl.ds(base, BPW)], idx_v, isem)

    @pl.when(sid == 0)
    def _load_table():
        pltpu.sync_copy(table_hbm, tshared)

    plsc.subcore_barrier()
    icp.wait()
    gcps = [
        pltpu.async_copy(
            tshared.at[idx_v.at[pl.ds(j * CHUNK, CHUNK)]], rows_v.at[j], gsems[j]
        )
        for j in range(NCHUNK)
    ]
    scps = []
    for j in range(NCHUNK):
        gcps[j].wait()
        scps.append(
            pltpu.async_copy(
                rows_v.at[j], out_hbm.at[pl.ds(base + j * CHUNK, CHUNK)], ssem
            )
        )
    for cp in scps:
        cp.wait()


def kernel(t, table):
    return _sc_gather2(t, table)
